# baseline (speedup 1.0000x reference)
import functools

import jax
import jax.numpy as jnp
from jax import lax
from jax.experimental import pallas as pl
from jax.experimental.pallas import tpu as pltpu
from jax.experimental.pallas import tpu_sc as plsc

BATCH = 16384
D = 64
L = 16
NC = 2
NS = 16
NW = NC * NS
CHUNK = BATCH // NW
SLAB = 256
NSLAB_MAIN = 3904 // NW
SENT = 0x7FFF0000
DUMMY = BATCH
STG = 128

_mesh = plsc.VectorSubcoreMesh(core_axis_name="c", subcore_axis_name="s")
_params = pltpu.CompilerParams(needs_layout_passes=False)

_i16 = lambda v: jnp.full((L,), v, jnp.int32)


@functools.partial(
    pl.kernel,
    mesh=_mesh,
    compiler_params=_params,
    out_type=(jax.ShapeDtypeStruct((BATCH + L, 128), jnp.float32),
              jax.ShapeDtypeStruct((BATCH + L, 128), jnp.float32)),
    scratch_types=[
        pltpu.VMEM((2048,), jnp.int32),
        pltpu.VMEM((BATCH + L, ), jnp.int32),
        pltpu.VMEM((BATCH + L, ), jnp.int32),
        pltpu.VMEM((BATCH + 2 * L,), jnp.int32),
        pltpu.VMEM((BATCH + 2 * L,), jnp.int32),
        pltpu.VMEM((2, D, SLAB), jnp.float32),
        pltpu.VMEM((STG, 128), jnp.float32),
        pltpu.VMEM((STG,), jnp.int32),
        pltpu.SemaphoreType.DMA,
        pltpu.SemaphoreType.DMA,
        pltpu.SemaphoreType.DMA,
    ],
)
def _gather_relu_sc(uidx_hbm, iidx_hbm, utabT_hbm, itabT_hbm,
                    out_u_hbm, out_i_hbm,
                    chunk_v, lidx_v, lpos_v, sidx_v, spos_v,
                    slab_v, stage_v, spost_v, semA, semB, semS):
    wid = lax.axis_index("s") * NC + lax.axis_index("c")
    iota16 = lax.iota(jnp.int32, L)
    c16 = [iota16 + _i16(16 * k) for k in range(D // L)]
    zero = jnp.zeros((L,), jnp.float32)

    for idx_hbm, tab_hbm, out_hbm in ((uidx_hbm, utabT_hbm, out_u_hbm),
                                      (iidx_hbm, itabT_hbm, out_i_hbm)):
        def chunk_body(ch, loff):
            pltpu.sync_copy(idx_hbm.at[pl.ds(ch * 2048, 2048)], chunk_v)

            def vec_body(j, off):
                v = chunk_v[pl.ds(j * L, L)]
                m = ((v // SLAB) % NW) == wid
                plsc.store_compressed(lidx_v.at[pl.ds(off, L)], v, mask=m)
                pos = ch * 2048 + j * L + iota16
                plsc.store_compressed(lpos_v.at[pl.ds(off, L)], pos, mask=m)
                return off + plsc.all_reduce_population_count(m)[0]

            return lax.fori_loop(0, 2048 // L, vec_body, loff)

        lcount = lax.fori_loop(0, BATCH // 2048, chunk_body, jnp.int32(0))
        lidx_v[pl.ds(lcount, L)] = jnp.full((L,), SENT, jnp.int32)
        lpos_v[pl.ds(lcount, L)] = jnp.full((L,), DUMMY, jnp.int32)
        ltrips = (lcount + L - 1) // L

        for q in range(STG // L):
            spost_v[pl.ds(q * L, L)] = jnp.full((L,), DUMMY, jnp.int32)

        def fetch(S, slot, sem):
            return pltpu.make_async_copy(
                tab_hbm.at[pl.ds(0, D),
                           pl.ds(pl.multiple_of(S * SLAB, 128), SLAB)],
                slab_v.at[slot], sem)

        def flush():
            pltpu.make_async_copy(stage_v, out_hbm.at[spost_v], semS).start()
            pltpu.make_async_copy(stage_v, out_hbm.at[spost_v], semS).wait()
            for q in range(STG // L):
                spost_v[pl.ds(q * L, L)] = jnp.full((L,), DUMMY, jnp.int32)

        def do_slab(S, slot, sc0, bias=0):
            lo = S * SLAB

            def filt(j, off):
                v = lidx_v[pl.ds(j * L, L)]
                p = lpos_v[pl.ds(j * L, L)]
                m = (v >= lo) & (v < lo + SLAB)
                plsc.store_compressed(sidx_v.at[pl.ds(off, L)], v, mask=m)
                plsc.store_compressed(spos_v.at[pl.ds(off, L)], p, mask=m)
                return off + plsc.all_reduce_population_count(m)[0]

            scount = lax.fori_loop(0, ltrips, filt, jnp.int32(0))
            sidx_v[pl.ds(scount, L)] = jnp.full((L,), lo, jnp.int32)
            spos_v[pl.ds(scount, L)] = jnp.full((L,), DUMMY, jnp.int32)

            def proc(j2, sc):
                sv = sidx_v[pl.ds(j2 * L, L)]
                pv = spos_v[pl.ds(j2 * L, L)]
                spost_v[pl.ds(sc, L)] = pv
                for e in range(L):
                    lane = _i16(bias) + ((sv[e] - lo) & (SLAB - 1))
                    for k in range(D // L):
                        vk = plsc.load_gather(slab_v.at[slot], [c16[k], lane])
                        stage_v[sc + e, pl.ds(16 * k, L)] = jnp.maximum(vk, 0.0)
                sc = sc + L

                @pl.when(sc >= STG)
                def _():
                    flush()

                return jnp.where(sc >= STG, jnp.int32(0), sc)

            return lax.fori_loop(0, (scount + L - 1) // L, proc, sc0)

        fetch(wid, 0, semA).start()

        def slab_body(t, sc):
            s0 = wid + NW * (2 * t)
            s1 = wid + NW * (2 * t + 1)
            fetch(s1, 1, semB).start()
            fetch(s0, 0, semA).wait()
            sc = do_slab(s0, 0, sc)

            @pl.when(t + 1 < NSLAB_MAIN // 2)
            def _():
                fetch(s0 + 2 * NW, 0, semA).start()

            fetch(s1, 1, semB).wait()
            return do_slab(s1, 1, sc)

        sc = lax.fori_loop(0, NSLAB_MAIN // 2, slab_body, jnp.int32(0))

        @pl.when(wid == 0)
        def _():
            pltpu.sync_copy(tab_hbm.at[pl.ds(0, D), pl.ds(3904 * SLAB, SLAB)],
                            slab_v.at[0])
            do_slab(jnp.int32(3904), 0, sc)
            flush()

        @pl.when(wid == 1)
        def _():
            pltpu.sync_copy(tab_hbm.at[pl.ds(0, D), pl.ds(3905 * SLAB, SLAB)],
                            slab_v.at[0])
            do_slab(jnp.int32(3905), 0, sc)
            flush()

        @pl.when(wid == 2)
        def _():
            toff = pl.multiple_of(jnp.int32(3906) * SLAB, 128)
            pltpu.sync_copy(tab_hbm.at[pl.ds(0, D), pl.ds(toff, 128)],
                            slab_v.at[0, pl.ds(0, D), pl.ds(0, 128)])
            do_slab(jnp.int32(3906), 0, sc)
            flush()

        @pl.when(wid > 2)
        def _():
            flush()


@functools.partial(
    pl.kernel,
    mesh=_mesh,
    compiler_params=_params,
    out_type=jax.ShapeDtypeStruct((BATCH,), jnp.float32),
    scratch_types=[
        pltpu.VMEM((128, 128), jnp.float32),
        pltpu.VMEM((128, 128), jnp.float32),
        pltpu.VMEM((CHUNK,), jnp.float32),
        pltpu.SemaphoreType.DMA,
    ],
)
def _dot_head_sc(ru_hbm, ri_hbm, out_hbm, ub_v, ib_v, res_v, sem):
    wid = lax.axis_index("s") * NC + lax.axis_index("c")
    base = wid * CHUNK
    iota16 = lax.iota(jnp.int32, L)
    zero = jnp.zeros((L,), jnp.float32)

    def blk_body(b, _):
        off = base + b * 128
        cu = pltpu.async_copy(ru_hbm.at[pl.ds(off, 128)], ub_v, sem)
        ci = pltpu.async_copy(ri_hbm.at[pl.ds(off, 128)], ib_v, sem)
        cu.wait()
        ci.wait()

        def grp(g, _2):
            out16 = zero
            for e in range(L):
                acc = zero
                for k in range(D // L):
                    uv = ub_v[g * L + e, pl.ds(16 * k, L)]
                    iv = ib_v[g * L + e, pl.ds(16 * k, L)]
                    acc = acc + uv * iv
                out16 = jnp.where(iota16 == e, jnp.sum(acc), out16)
            res_v[pl.ds(b * 128 + g * L, L)] = 4.0 / (1.0 + jnp.exp(-out16)) + 1.0
            return 0

        lax.fori_loop(0, 128 // L, grp, 0)
        return 0

    lax.fori_loop(0, CHUNK // 128, blk_body, 0)
    pltpu.sync_copy(res_v, out_hbm.at[pl.ds(base, CHUNK)])


def kernel(user_indices, item_indices, user_table, item_table):
    ru, ri = _gather_relu_sc(user_indices.astype(jnp.int32),
                             item_indices.astype(jnp.int32),
                             user_table.T, item_table.T)
    return _dot_head_sc(ru, ri)
